# Initial kernel scaffold; baseline (speedup 1.0000x reference)
import jax
import jax.numpy as jnp
from jax.experimental import pallas as pl


def kernel(x, ln_g, ln_b, W1, b1, W2, b2):
    raise NotImplementedError("write your pallas kernel here")


# speedup vs baseline: 1.7590x; 1.7590x over previous
import functools

import jax
import jax.numpy as jnp
from jax.experimental import pallas as pl
from jax.experimental.pallas import tpu as pltpu

T = 8192
IN_DIM = 4096
HID = 4096
E = 64

TB = 512
HB = 512


def _body(x_ref, g_ref, b_ref, w1_ref, b1_ref, w2_ref, b2_ref, o_ref,
          xn_ref, acc_ref):
    j = pl.program_id(1)

    @pl.when(j == 0)
    def _():
        xv = x_ref[...]
        mu = jnp.mean(xv, axis=1, keepdims=True)
        xc = xv - mu
        var = jnp.mean(xc * xc, axis=1, keepdims=True)
        xn_ref[...] = xc * jax.lax.rsqrt(var + 1e-5) * g_ref[...] + b_ref[...]
        acc_ref[...] = jnp.zeros_like(acc_ref)

    h = jnp.dot(xn_ref[...], w1_ref[...],
                preferred_element_type=jnp.float32,
                precision=jax.lax.Precision.DEFAULT) + b1_ref[...]
    h = 0.5 * h * (1.0 + jax.lax.erf(h * 0.7071067811865476))
    acc_ref[...] += jnp.dot(h, w2_ref[...],
                            preferred_element_type=jnp.float32,
                            precision=jax.lax.Precision.DEFAULT)

    @pl.when(j == pl.num_programs(1) - 1)
    def _():
        l = acc_ref[...] + b2_ref[...]
        idx = jax.lax.broadcasted_iota(jnp.int32, l.shape, 1)
        m1 = jnp.max(l, axis=1, keepdims=True)
        a1 = jnp.min(jnp.where(l == m1, idx, E), axis=1, keepdims=True)
        l2 = jnp.where(idx == a1, -jnp.inf, l)
        m2 = jnp.max(l2, axis=1, keepdims=True)
        a2 = jnp.min(jnp.where(l2 == m2, idx, E), axis=1, keepdims=True)
        mask = (idx == a1) | (idx == a2)
        z = 1.0 + jnp.exp(m2 - m1)
        o_ref[...] = jnp.where(mask, jnp.exp(l - m1) / z, 0.0)


@functools.partial(jax.jit, static_argnames=("interpret",))
def kernel(x, ln_g, ln_b, W1, b1, W2, b2, interpret=False):
    g2 = ln_g.reshape(1, IN_DIM)
    b2d = ln_b.reshape(1, IN_DIM)
    b1_2 = b1.reshape(1, HID)
    b2_2 = b2.reshape(1, E)
    grid = (T // TB, HID // HB)
    out = pl.pallas_call(
        _body,
        grid=grid,
        in_specs=[
            pl.BlockSpec((TB, IN_DIM), lambda i, j: (i, 0)),
            pl.BlockSpec((1, IN_DIM), lambda i, j: (0, 0)),
            pl.BlockSpec((1, IN_DIM), lambda i, j: (0, 0)),
            pl.BlockSpec((IN_DIM, HB), lambda i, j: (0, j)),
            pl.BlockSpec((1, HB), lambda i, j: (0, j)),
            pl.BlockSpec((HB, E), lambda i, j: (j, 0)),
            pl.BlockSpec((1, E), lambda i, j: (0, 0)),
        ],
        out_specs=pl.BlockSpec((TB, E), lambda i, j: (i, 0)),
        out_shape=jax.ShapeDtypeStruct((T, E), jnp.float32),
        scratch_shapes=[
            pltpu.VMEM((TB, IN_DIM), jnp.float32),
            pltpu.VMEM((TB, E), jnp.float32),
        ],
        compiler_params=pltpu.CompilerParams(
            dimension_semantics=("parallel", "arbitrary"),
        ),
        interpret=interpret,
    )(x, g2, b2d, W1, b1_2, W2, b2_2)
    return out


# speedup vs baseline: 1.8355x; 1.0435x over previous
import functools

import jax
import jax.numpy as jnp
from jax.experimental import pallas as pl
from jax.experimental.pallas import tpu as pltpu

T = 8192
IN_DIM = 4096
HID = 4096
E = 64

TB = 512
HB = 512


def _body(x_ref, g_ref, b_ref, w1_ref, b1_ref, w2_ref, b2_ref, o_ref,
          xn_ref, acc_ref):
    j = pl.program_id(1)

    @pl.when(j == 0)
    def _():
        xv = x_ref[...]
        mu = jnp.mean(xv, axis=1, keepdims=True)
        xc = xv - mu
        var = jnp.mean(xc * xc, axis=1, keepdims=True)
        xn = xc * jax.lax.rsqrt(var + 1e-5) * g_ref[...] + b_ref[...]
        xn_ref[...] = xn.astype(jnp.bfloat16)
        acc_ref[...] = jnp.zeros_like(acc_ref)

    h = jnp.dot(xn_ref[...], w1_ref[...],
                preferred_element_type=jnp.float32) + b1_ref[...]
    h = 0.5 * h * (1.0 + jax.lax.erf(h * 0.7071067811865476))
    acc_ref[...] += jnp.dot(h.astype(jnp.bfloat16), w2_ref[...],
                            preferred_element_type=jnp.float32)

    @pl.when(j == pl.num_programs(1) - 1)
    def _():
        l = acc_ref[...] + b2_ref[...]
        idx = jax.lax.broadcasted_iota(jnp.int32, l.shape, 1)
        m1 = jnp.max(l, axis=1, keepdims=True)
        a1 = jnp.min(jnp.where(l == m1, idx, E), axis=1, keepdims=True)
        l2 = jnp.where(idx == a1, -jnp.inf, l)
        m2 = jnp.max(l2, axis=1, keepdims=True)
        a2 = jnp.min(jnp.where(l2 == m2, idx, E), axis=1, keepdims=True)
        mask = (idx == a1) | (idx == a2)
        z = 1.0 + jnp.exp(m2 - m1)
        o_ref[...] = jnp.where(mask, jnp.exp(l - m1) / z, 0.0)


@functools.partial(jax.jit, static_argnames=("interpret",))
def kernel(x, ln_g, ln_b, W1, b1, W2, b2, interpret=False):
    g2 = ln_g.reshape(1, IN_DIM)
    b2d = ln_b.reshape(1, IN_DIM)
    b1_2 = b1.reshape(1, HID)
    b2_2 = b2.reshape(1, E)
    w1_bf = W1.astype(jnp.bfloat16)
    w2_bf = W2.astype(jnp.bfloat16)
    grid = (T // TB, HID // HB)
    out = pl.pallas_call(
        _body,
        grid=grid,
        in_specs=[
            pl.BlockSpec((TB, IN_DIM), lambda i, j: (i, 0)),
            pl.BlockSpec((1, IN_DIM), lambda i, j: (0, 0)),
            pl.BlockSpec((1, IN_DIM), lambda i, j: (0, 0)),
            pl.BlockSpec((IN_DIM, HB), lambda i, j: (0, j)),
            pl.BlockSpec((1, HB), lambda i, j: (0, j)),
            pl.BlockSpec((HB, E), lambda i, j: (j, 0)),
            pl.BlockSpec((1, E), lambda i, j: (0, 0)),
        ],
        out_specs=pl.BlockSpec((TB, E), lambda i, j: (i, 0)),
        out_shape=jax.ShapeDtypeStruct((T, E), jnp.float32),
        scratch_shapes=[
            pltpu.VMEM((TB, IN_DIM), jnp.bfloat16),
            pltpu.VMEM((TB, E), jnp.float32),
        ],
        compiler_params=pltpu.CompilerParams(
            dimension_semantics=("parallel", "arbitrary"),
        ),
        interpret=interpret,
    )(x, g2, b2d, w1_bf, b1_2, w2_bf, b2_2)
    return out
